# Initial kernel scaffold; baseline (speedup 1.0000x reference)
import jax
import jax.numpy as jnp
from jax.experimental import pallas as pl


def kernel(x, rbf, sbf, idx_kj, idx_ji, params):
    raise NotImplementedError("write your pallas kernel here")


# speedup vs baseline: 3.8114x; 3.8114x over previous
import functools

import jax
import jax.numpy as jnp
from jax import lax
from jax.experimental import pallas as pl
from jax.experimental.pallas import tpu as pltpu
from jax.experimental.pallas import tpu_sc as plsc

E = 160000
T = 640000
H = 256
INT = 64

BE = 2000
BT = 4000

NC = 2
NS = 16
NW = NC * NS

CH = 400
NCHUNK = T // CH
CPW = NCHUNK // NW
CPT = NCHUNK // NS

R = 26688
TROWS = R // NS
ZR = 139
PASSES = 3
CHS = 160
NCHUNK_S = T // CHS
CPT = NCHUNK_S // NS


def _silu(v):
    return v * jax.nn.sigmoid(v)


def _front_body(x_ref, rbf_ref, wji_ref, bji_ref, wkj_ref, bkj_ref,
                rw1_ref, rw2_ref, wdown_ref, xji_ref, xd_ref):
    x = x_ref[...]
    xji_ref[...] = _silu(jnp.dot(x, wji_ref[...]) + bji_ref[...])
    rbft = jnp.dot(jnp.dot(rbf_ref[...], rw1_ref[...]), rw2_ref[...])
    t = _silu(jnp.dot(x, wkj_ref[...]) + bkj_ref[...]) * rbft
    xd_ref[...] = _silu(jnp.dot(t, wdown_ref[...]))


def _run_front(x, rbf, p):
    nr = rbf.shape[1]
    grid = E // BE
    full = lambda s: pl.BlockSpec(s, lambda i: (0, 0))
    return pl.pallas_call(
        _front_body,
        grid=(grid,),
        in_specs=[
            pl.BlockSpec((BE, H), lambda i: (i, 0)),
            pl.BlockSpec((BE, nr), lambda i: (i, 0)),
            full((H, H)), full((1, H)), full((H, H)), full((1, H)),
            full((nr, 8)), full((8, H)), full((H, INT)),
        ],
        out_specs=[
            pl.BlockSpec((BE, H), lambda i: (i, 0)),
            pl.BlockSpec((BE, INT), lambda i: (i, 0)),
        ],
        out_shape=[
            jax.ShapeDtypeStruct((E, H), jnp.float32),
            jax.ShapeDtypeStruct((E, INT), jnp.float32),
        ],
    )(x, rbf, p['wji'], p['bji'].reshape(1, H), p['wkj'], p['bkj'].reshape(1, H),
      p['rbf_w1'], p['rbf_w2'], p['wdown'])


def _sbf_body(sbf_ref, w1_ref, w2_ref, out_ref):
    out_ref[...] = jnp.dot(jnp.dot(sbf_ref[...], w1_ref[...]), w2_ref[...])


def _run_sbf(sbf, p):
    ns = sbf.shape[1]
    grid = T // BT
    return pl.pallas_call(
        _sbf_body,
        grid=(grid,),
        in_specs=[
            pl.BlockSpec((BT, ns), lambda i: (i, 0)),
            pl.BlockSpec((ns, 8), lambda i: (0, 0)),
            pl.BlockSpec((8, INT), lambda i: (0, 0)),
        ],
        out_specs=pl.BlockSpec((BT, INT), lambda i: (i, 0)),
        out_shape=jax.ShapeDtypeStruct((T, INT), jnp.float32),
    )(sbf, p['sbf_w1'], p['sbf_w2'])


def _sc_gather_mul(xd, sbf_t, idx2d):
    mesh = plsc.VectorSubcoreMesh(core_axis_name="c", subcore_axis_name="s")

    @functools.partial(
        pl.kernel,
        out_type=jax.ShapeDtypeStruct((T, INT), jnp.float32),
        mesh=mesh,
        compiler_params=pltpu.CompilerParams(use_tc_tiling_on_sc=False),
        scratch_types=[
            pltpu.VMEM((5, 80), jnp.int32),
            pltpu.VMEM((CH, INT), jnp.float32),
            pltpu.VMEM((CH, INT), jnp.float32),
            pltpu.SemaphoreType.DMA,
        ],
    )
    def k(xd_hbm, sbf_hbm, idx_hbm, g_hbm, idx_v, rows_v, sbf_v, sem):
        wid = lax.axis_index("s") * NC + lax.axis_index("c")

        def chunk(i, carry):
            ch = wid * CPW + i
            off = ch * CH
            pltpu.sync_copy(idx_hbm.at[pl.ds(ch * 5, 5)], idx_v)
            cps = [
                pltpu.async_copy(xd_hbm.at[idx_v.at[j]],
                                 rows_v.at[pl.ds(j * 80, 80)], sem)
                for j in range(5)
            ]
            pltpu.sync_copy(sbf_hbm.at[pl.ds(off, CH)], sbf_v)
            for cp in cps:
                cp.wait()

            def mulrow(r, c2):
                for j in range(INT // 16):
                    sl = pl.ds(j * 16, 16)
                    rows_v[r, sl] = rows_v[r, sl] * sbf_v[r, sl]
                return c2

            lax.fori_loop(0, CH, mulrow, 0)
            pltpu.sync_copy(rows_v, g_hbm.at[pl.ds(off, CH)])
            return carry

        lax.fori_loop(0, CPW, chunk, 0)

    return k(xd, sbf_t, idx2d)


def _sc_scatter(g, idx2d):
    mesh = plsc.VectorSubcoreMesh(core_axis_name="c", subcore_axis_name="s")

    @functools.partial(
        pl.kernel,
        out_type=jax.ShapeDtypeStruct((6 * R, INT), jnp.float32),
        mesh=mesh,
        compiler_params=pltpu.CompilerParams(use_tc_tiling_on_sc=False),
        scratch_types=[
            pltpu.VMEM((ZR, INT), jnp.float32),
            pltpu.VMEM((2, 80), jnp.int32),
            pltpu.VMEM((2, 80), jnp.int32),
            pltpu.VMEM((CHS, INT), jnp.float32),
            pltpu.VMEM_SHARED((R + 8, INT), jnp.float32),
            pltpu.SemaphoreType.DMA,
        ],
    )
    def k(g_hbm, j_hbm, out_hbm, stage, jbuf, libuf, vbuf, acc, sem):
        c = lax.axis_index("c")
        s = lax.axis_index("s")
        row0 = s * TROWS
        zv = jnp.zeros((16,), jnp.float32)
        for p in range(PASSES):
            base = (2 * p) * R + c * R

            def zrow(r, carry):
                for j in range(INT // 16):
                    stage[r, pl.ds(j * 16, 16)] = zv
                return carry

            lax.fori_loop(0, ZR, zrow, 0)
            for i in range(TROWS // ZR):
                pltpu.sync_copy(stage, acc.at[pl.ds(row0 + i * ZR, ZR)])
            plsc.subcore_barrier()

            def chunk(i, carry):
                ch = s * CPT + i
                off = ch * CHS
                pltpu.sync_copy(j_hbm.at[pl.ds(ch * 2, 2)], jbuf)
                cp = pltpu.async_copy(g_hbm.at[pl.ds(off, CHS)], vbuf, sem)
                for a in range(2):
                    for b in range(5):
                        sl = pl.ds(b * 16, 16)
                        li = jbuf[a, sl] - base
                        ok = (li >= 0) & (li < R)
                        libuf[a, sl] = jnp.where(ok, li, R)
                cp.wait()
                cps = [
                    pltpu.async_copy(vbuf.at[pl.ds(a * 80, 80)],
                                     acc.at[libuf.at[a]], sem, add=True)
                    for a in range(2)
                ]
                for cp2 in cps:
                    cp2.wait()
                return carry

            lax.fori_loop(0, CPT, chunk, 0)
            plsc.subcore_barrier()
            for i in range(TROWS // ZR):
                r0 = row0 + i * ZR
                pltpu.sync_copy(acc.at[pl.ds(r0, ZR)], stage)
                pltpu.sync_copy(stage, out_hbm.at[pl.ds(base + r0, ZR)])

    return k(g, idx2d)


def _back_body(seg_ref, xji_ref, x_ref, wup_ref,
               b0w1_ref, b0b1_ref, b0w2_ref, b0b2_ref,
               linw_ref, linb_ref,
               a0w1_ref, a0b1_ref, a0w2_ref, a0b2_ref,
               a1w1_ref, a1b1_ref, a1w2_ref, a1b2_ref,
               out_ref):
    u = _silu(jnp.dot(seg_ref[...], wup_ref[...]))
    h = xji_ref[...] + u
    h = h + _silu(jnp.dot(_silu(jnp.dot(h, b0w1_ref[...]) + b0b1_ref[...]),
                          b0w2_ref[...]) + b0b2_ref[...])
    h = _silu(jnp.dot(h, linw_ref[...]) + linb_ref[...]) + x_ref[...]
    h = h + _silu(jnp.dot(_silu(jnp.dot(h, a0w1_ref[...]) + a0b1_ref[...]),
                          a0w2_ref[...]) + a0b2_ref[...])
    h = h + _silu(jnp.dot(_silu(jnp.dot(h, a1w1_ref[...]) + a1b1_ref[...]),
                          a1w2_ref[...]) + a1b2_ref[...])
    out_ref[...] = h


def _run_back(seg, xji, x, p):
    grid = E // BE
    full = lambda s: pl.BlockSpec(s, lambda i: (0, 0))
    w = lambda: full((H, H))
    b = lambda: full((1, H))
    return pl.pallas_call(
        _back_body,
        grid=(grid,),
        in_specs=[
            pl.BlockSpec((BE, INT), lambda i: (i, 0)),
            pl.BlockSpec((BE, H), lambda i: (i, 0)),
            pl.BlockSpec((BE, H), lambda i: (i, 0)),
            full((INT, H)),
            w(), b(), w(), b(),
            w(), b(),
            w(), b(), w(), b(),
            w(), b(), w(), b(),
        ],
        out_specs=pl.BlockSpec((BE, H), lambda i: (i, 0)),
        out_shape=jax.ShapeDtypeStruct((E, H), jnp.float32),
    )(seg, xji, x, p['wup'],
      p['res_b0_w1'], p['res_b0_b1'].reshape(1, H), p['res_b0_w2'], p['res_b0_b2'].reshape(1, H),
      p['lin_w'], p['lin_b'].reshape(1, H),
      p['res_a0_w1'], p['res_a0_b1'].reshape(1, H), p['res_a0_w2'], p['res_a0_b2'].reshape(1, H),
      p['res_a1_w1'], p['res_a1_b1'].reshape(1, H), p['res_a1_w2'], p['res_a1_b2'].reshape(1, H))


def kernel(x, rbf, sbf, idx_kj, idx_ji, params):
    p = params
    xji, xd = _run_front(x, rbf, p)
    sbf_t = _run_sbf(sbf, p)
    idxkj2 = idx_kj.astype(jnp.int32).reshape(T // 80, 80)
    idxji2 = idx_ji.astype(jnp.int32).reshape(T // 80, 80)
    g = _sc_gather_mul(xd, sbf_t, idxkj2)
    segp = _sc_scatter(g, idxji2)
    return _run_back(segp, xji, x, p)


# speedup vs baseline: 3.9988x; 1.0492x over previous
import functools

import jax
import jax.numpy as jnp
from jax import lax
from jax.experimental import pallas as pl
from jax.experimental.pallas import tpu as pltpu
from jax.experimental.pallas import tpu_sc as plsc

E = 160000
T = 640000
H = 256
INT = 64

BE = 2000
BT = 4000

NC = 2
NS = 16
NW = NC * NS

CH = 400
NCHUNK = T // CH
CPW = NCHUNK // NW

QN = 4
QW = INT // QN
RQ = 80000
TROWSQ = RQ // NS
ZRQ = 1000
CHS = 400
NCHUNK_S = T // CHS
CPT = NCHUNK_S // NS


def _silu(v):
    return v * jax.nn.sigmoid(v)


def _front_body(x_ref, rbf_ref, wji_ref, bji_ref, wkj_ref, bkj_ref,
                rw1_ref, rw2_ref, wdown_ref, xji_ref, xd_ref):
    x = x_ref[...]
    xji_ref[...] = _silu(jnp.dot(x, wji_ref[...]) + bji_ref[...])
    rbft = jnp.dot(jnp.dot(rbf_ref[...], rw1_ref[...]), rw2_ref[...])
    t = _silu(jnp.dot(x, wkj_ref[...]) + bkj_ref[...]) * rbft
    xd_ref[...] = _silu(jnp.dot(t, wdown_ref[...]))


def _run_front(x, rbf, p):
    nr = rbf.shape[1]
    grid = E // BE
    full = lambda s: pl.BlockSpec(s, lambda i: (0, 0))
    return pl.pallas_call(
        _front_body,
        grid=(grid,),
        in_specs=[
            pl.BlockSpec((BE, H), lambda i: (i, 0)),
            pl.BlockSpec((BE, nr), lambda i: (i, 0)),
            full((H, H)), full((1, H)), full((H, H)), full((1, H)),
            full((nr, 8)), full((8, H)), full((H, INT)),
        ],
        out_specs=[
            pl.BlockSpec((BE, H), lambda i: (i, 0)),
            pl.BlockSpec((BE, INT), lambda i: (i, 0)),
        ],
        out_shape=[
            jax.ShapeDtypeStruct((E, H), jnp.float32),
            jax.ShapeDtypeStruct((E, INT), jnp.float32),
        ],
    )(x, rbf, p['wji'], p['bji'].reshape(1, H), p['wkj'], p['bkj'].reshape(1, H),
      p['rbf_w1'], p['rbf_w2'], p['wdown'])


def _sbf_body(sbf_ref, w1_ref, w2_ref, out_ref):
    out_ref[...] = jnp.dot(jnp.dot(sbf_ref[...], w1_ref[...]), w2_ref[...])


def _run_sbf(sbf, p):
    ns = sbf.shape[1]
    grid = T // BT
    return pl.pallas_call(
        _sbf_body,
        grid=(grid,),
        in_specs=[
            pl.BlockSpec((BT, ns), lambda i: (i, 0)),
            pl.BlockSpec((ns, 8), lambda i: (0, 0)),
            pl.BlockSpec((8, INT), lambda i: (0, 0)),
        ],
        out_specs=pl.BlockSpec((BT, INT), lambda i: (i, 0)),
        out_shape=jax.ShapeDtypeStruct((T, INT), jnp.float32),
    )(sbf, p['sbf_w1'], p['sbf_w2'])


def _sc_gather_mul(xd, sbf_t, idx2d):
    mesh = plsc.VectorSubcoreMesh(core_axis_name="c", subcore_axis_name="s")

    @functools.partial(
        pl.kernel,
        out_type=jax.ShapeDtypeStruct((QN, T, QW), jnp.float32),
        mesh=mesh,
        compiler_params=pltpu.CompilerParams(use_tc_tiling_on_sc=False),
        scratch_types=[
            pltpu.VMEM((2, 5, 80), jnp.int32),
            pltpu.VMEM((2, CH, INT), jnp.float32),
            pltpu.VMEM((2, CH, INT), jnp.float32),
            pltpu.SemaphoreType.DMA, pltpu.SemaphoreType.DMA,
            pltpu.SemaphoreType.DMA, pltpu.SemaphoreType.DMA,
            pltpu.SemaphoreType.DMA, pltpu.SemaphoreType.DMA,
        ],
    )
    def k(xd_hbm, sbf_hbm, idx_hbm, g_hbm, idx_v, rows_v, sbf_v,
          semg0, semg1, sems0, sems1, semw0, semw1):
        semg = (semg0, semg1)
        sems = (sems0, sems1)
        semw = (semw0, semw1)
        wid = lax.axis_index("s") * NC + lax.axis_index("c")
        c0 = wid * CPW

        def load(ci, b):
            pltpu.sync_copy(idx_hbm.at[pl.ds(ci * 5, 5)], idx_v.at[b])
            for j in range(5):
                pltpu.async_copy(xd_hbm.at[idx_v.at[b, j]],
                                 rows_v.at[b, pl.ds(j * 80, 80)], semg[b])
            pltpu.async_copy(sbf_hbm.at[pl.ds(ci * CH, CH)], sbf_v.at[b], sems[b])

        def drain_w(b):
            for q in range(QN):
                pltpu.make_async_copy(
                    rows_v.at[b, :, pl.ds(q * QW, QW)],
                    g_hbm.at[q, pl.ds(0, CH)], semw[b]).wait()

        def process(ci, b):
            for j in range(5):
                pltpu.make_async_copy(
                    xd_hbm.at[pl.ds(0, 80)],
                    rows_v.at[b, pl.ds(j * 80, 80)], semg[b]).wait()
            pltpu.make_async_copy(
                sbf_hbm.at[pl.ds(0, CH)], sbf_v.at[b], sems[b]).wait()

            def mulrow(r, c2):
                for j in range(INT // 16):
                    sl = pl.ds(j * 16, 16)
                    rows_v[b, r, sl] = rows_v[b, r, sl] * sbf_v[b, r, sl]
                return c2

            lax.fori_loop(0, CH, mulrow, 0)
            off = ci * CH
            for q in range(QN):
                pltpu.async_copy(rows_v.at[b, :, pl.ds(q * QW, QW)],
                                 g_hbm.at[q, pl.ds(off, CH)], semw[b])

        load(c0, 0)
        npairs = CPW // 2

        def pair(i, carry):
            @pl.when(i > 0)
            def _():
                drain_w(1)
            load(c0 + 2 * i + 1, 1)
            process(c0 + 2 * i, 0)

            @pl.when(i + 1 < npairs)
            def _():
                drain_w(0)
                load(c0 + 2 * i + 2, 0)
            process(c0 + 2 * i + 1, 1)
            return carry

        lax.fori_loop(0, npairs, pair, 0)
        drain_w(0)
        drain_w(1)

    return k(xd, sbf_t, idx2d)


def _sc_scatter(g, idx2d):
    mesh = plsc.VectorSubcoreMesh(core_axis_name="c", subcore_axis_name="s")

    @functools.partial(
        pl.kernel,
        out_type=jax.ShapeDtypeStruct((E, INT), jnp.float32),
        mesh=mesh,
        compiler_params=pltpu.CompilerParams(use_tc_tiling_on_sc=False),
        scratch_types=[
            pltpu.VMEM((ZRQ, QW), jnp.float32),
            pltpu.VMEM((ZRQ, QW), jnp.float32),
            pltpu.VMEM((2, 5, 80), jnp.int32),
            pltpu.VMEM((2, 5, 80), jnp.int32),
            pltpu.VMEM((2, CHS, QW), jnp.float32),
            pltpu.VMEM_SHARED((RQ + 8, QW), jnp.float32),
            pltpu.SemaphoreType.DMA, pltpu.SemaphoreType.DMA,
            pltpu.SemaphoreType.DMA, pltpu.SemaphoreType.DMA,
        ],
    )
    def k(g_hbm, j_hbm, out_hbm, zbuf, stage, jbuf, libuf, vbuf, acc,
          semv0, semv1, sema0, sema1):
        semv = (semv0, semv1)
        sema = (sema0, sema1)
        c = lax.axis_index("c")
        s = lax.axis_index("s")
        crow = c * RQ
        row0 = s * TROWSQ
        zv = jnp.zeros((16,), jnp.float32)

        def zrow(r, carry):
            zbuf[r, pl.ds(0, QW)] = zv
            return carry

        lax.fori_loop(0, ZRQ, zrow, 0)

        for q in range(QN):
            for i in range(TROWSQ // ZRQ):
                pltpu.sync_copy(zbuf, acc.at[pl.ds(row0 + i * ZRQ, ZRQ)])
            plsc.subcore_barrier()

            def load(ci, b):
                off = ci * CHS
                pltpu.sync_copy(j_hbm.at[pl.ds(ci * 5, 5)], jbuf.at[b])
                pltpu.async_copy(g_hbm.at[q, pl.ds(off, CHS)], vbuf.at[b], semv[b])

            def drain_a(b):
                for a in range(5):
                    pltpu.make_async_copy(
                        vbuf.at[b, pl.ds(a * 80, 80)],
                        acc.at[pl.ds(0, 80)], sema[b]).wait()

            def process(ci, b):
                for a in range(5):
                    for bb in range(5):
                        sl = pl.ds(bb * 16, 16)
                        li = jbuf[b, a, sl] - crow
                        ok = (li >= 0) & (li < RQ)
                        libuf[b, a, sl] = jnp.where(ok, li, RQ)
                pltpu.make_async_copy(
                    g_hbm.at[q, pl.ds(0, CHS)], vbuf.at[b], semv[b]).wait()
                for a in range(5):
                    pltpu.async_copy(vbuf.at[b, pl.ds(a * 80, 80)],
                                     acc.at[libuf.at[b, a]], sema[b], add=True)

            ch0 = s * CPT
            load(ch0, 0)
            npairs = CPT // 2

            def pair(i, carry):
                @pl.when(i > 0)
                def _():
                    drain_a(1)
                load(ch0 + 2 * i + 1, 1)
                process(ch0 + 2 * i, 0)

                @pl.when(i + 1 < npairs)
                def _():
                    drain_a(0)
                    load(ch0 + 2 * i + 2, 0)
                process(ch0 + 2 * i + 1, 1)
                return carry

            lax.fori_loop(0, npairs, pair, 0)
            drain_a(0)
            drain_a(1)
            plsc.subcore_barrier()
            for i in range(TROWSQ // ZRQ):
                r0 = row0 + i * ZRQ
                pltpu.sync_copy(acc.at[pl.ds(r0, ZRQ)], stage)
                pltpu.sync_copy(
                    stage,
                    out_hbm.at[pl.ds(crow + r0, ZRQ), pl.ds(q * QW, QW)])

    return k(g, idx2d)


def _back_body(seg_ref, xji_ref, x_ref, wup_ref,
               b0w1_ref, b0b1_ref, b0w2_ref, b0b2_ref,
               linw_ref, linb_ref,
               a0w1_ref, a0b1_ref, a0w2_ref, a0b2_ref,
               a1w1_ref, a1b1_ref, a1w2_ref, a1b2_ref,
               out_ref):
    u = _silu(jnp.dot(seg_ref[...], wup_ref[...]))
    h = xji_ref[...] + u
    h = h + _silu(jnp.dot(_silu(jnp.dot(h, b0w1_ref[...]) + b0b1_ref[...]),
                          b0w2_ref[...]) + b0b2_ref[...])
    h = _silu(jnp.dot(h, linw_ref[...]) + linb_ref[...]) + x_ref[...]
    h = h + _silu(jnp.dot(_silu(jnp.dot(h, a0w1_ref[...]) + a0b1_ref[...]),
                          a0w2_ref[...]) + a0b2_ref[...])
    h = h + _silu(jnp.dot(_silu(jnp.dot(h, a1w1_ref[...]) + a1b1_ref[...]),
                          a1w2_ref[...]) + a1b2_ref[...])
    out_ref[...] = h


def _run_back(seg, xji, x, p):
    grid = E // BE
    full = lambda s: pl.BlockSpec(s, lambda i: (0, 0))
    w = lambda: full((H, H))
    b = lambda: full((1, H))
    return pl.pallas_call(
        _back_body,
        grid=(grid,),
        in_specs=[
            pl.BlockSpec((BE, INT), lambda i: (i, 0)),
            pl.BlockSpec((BE, H), lambda i: (i, 0)),
            pl.BlockSpec((BE, H), lambda i: (i, 0)),
            full((INT, H)),
            w(), b(), w(), b(),
            w(), b(),
            w(), b(), w(), b(),
            w(), b(), w(), b(),
        ],
        out_specs=pl.BlockSpec((BE, H), lambda i: (i, 0)),
        out_shape=jax.ShapeDtypeStruct((E, H), jnp.float32),
    )(seg, xji, x, p['wup'],
      p['res_b0_w1'], p['res_b0_b1'].reshape(1, H), p['res_b0_w2'], p['res_b0_b2'].reshape(1, H),
      p['lin_w'], p['lin_b'].reshape(1, H),
      p['res_a0_w1'], p['res_a0_b1'].reshape(1, H), p['res_a0_w2'], p['res_a0_b2'].reshape(1, H),
      p['res_a1_w1'], p['res_a1_b1'].reshape(1, H), p['res_a1_w2'], p['res_a1_b2'].reshape(1, H))


def kernel(x, rbf, sbf, idx_kj, idx_ji, params):
    p = params
    xji, xd = _run_front(x, rbf, p)
    sbf_t = _run_sbf(sbf, p)
    idxkj2 = idx_kj.astype(jnp.int32).reshape(T // 80, 80)
    idxji2 = idx_ji.astype(jnp.int32).reshape(T // 80, 80)
    g = _sc_gather_mul(xd, sbf_t, idxkj2)
    segp = _sc_scatter(g, idxji2)
    return _run_back(segp, xji, x, p)


# speedup vs baseline: 5.8841x; 1.4715x over previous
import functools

import jax
import jax.numpy as jnp
from jax import lax
from jax.experimental import pallas as pl
from jax.experimental.pallas import tpu as pltpu
from jax.experimental.pallas import tpu_sc as plsc

E = 160000
T = 640000
H = 256
INT = 64

BE = 2000
BT = 4000

NC = 2
NS = 16
NW = NC * NS

CH = 400
NCHUNK = T // CH
CPW = NCHUNK // NW

QN = 2
QW = INT // QN
RQ = 80000
TROWSQ = RQ // NS
ZRQ = 500
CHS = 800
NCHUNK_S = T // CHS
CPT = NCHUNK_S // NS
NJ = CHS // 80


def _silu(v):
    return v * jax.nn.sigmoid(v)


def _front_body(x_ref, rbf_ref, wji_ref, bji_ref, wkj_ref, bkj_ref,
                rw1_ref, rw2_ref, wdown_ref, xji_ref, xd_ref):
    x = x_ref[...]
    xji_ref[...] = _silu(jnp.dot(x, wji_ref[...]) + bji_ref[...])
    rbft = jnp.dot(jnp.dot(rbf_ref[...], rw1_ref[...]), rw2_ref[...])
    t = _silu(jnp.dot(x, wkj_ref[...]) + bkj_ref[...]) * rbft
    xd_ref[...] = _silu(jnp.dot(t, wdown_ref[...])).astype(jnp.bfloat16)


def _run_front(x, rbf, p):
    nr = rbf.shape[1]
    grid = E // BE
    full = lambda s: pl.BlockSpec(s, lambda i: (0, 0))
    return pl.pallas_call(
        _front_body,
        grid=(grid,),
        in_specs=[
            pl.BlockSpec((BE, H), lambda i: (i, 0)),
            pl.BlockSpec((BE, nr), lambda i: (i, 0)),
            full((H, H)), full((1, H)), full((H, H)), full((1, H)),
            full((nr, 8)), full((8, H)), full((H, INT)),
        ],
        out_specs=[
            pl.BlockSpec((BE, H), lambda i: (i, 0)),
            pl.BlockSpec((BE, INT), lambda i: (i, 0)),
        ],
        out_shape=[
            jax.ShapeDtypeStruct((E, H), jnp.float32),
            jax.ShapeDtypeStruct((E, INT), jnp.bfloat16),
        ],
    )(x, rbf, p['wji'], p['bji'].reshape(1, H), p['wkj'], p['bkj'].reshape(1, H),
      p['rbf_w1'], p['rbf_w2'], p['wdown'])


def _sbf_body(sbf_ref, w1_ref, w2_ref, out_ref):
    out_ref[...] = jnp.dot(jnp.dot(sbf_ref[...], w1_ref[...]),
                           w2_ref[...]).astype(jnp.bfloat16)


def _run_sbf(sbf, p):
    ns = sbf.shape[1]
    grid = T // BT
    return pl.pallas_call(
        _sbf_body,
        grid=(grid,),
        in_specs=[
            pl.BlockSpec((BT, ns), lambda i: (i, 0)),
            pl.BlockSpec((ns, 8), lambda i: (0, 0)),
            pl.BlockSpec((8, INT), lambda i: (0, 0)),
        ],
        out_specs=pl.BlockSpec((BT, INT), lambda i: (i, 0)),
        out_shape=jax.ShapeDtypeStruct((T, INT), jnp.bfloat16),
    )(sbf, p['sbf_w1'], p['sbf_w2'])


def _sc_gather_mul(xd, sbf_t, idx2d):
    mesh = plsc.VectorSubcoreMesh(core_axis_name="c", subcore_axis_name="s")

    @functools.partial(
        pl.kernel,
        out_type=jax.ShapeDtypeStruct((QN, T, QW), jnp.bfloat16),
        mesh=mesh,
        compiler_params=pltpu.CompilerParams(use_tc_tiling_on_sc=False),
        scratch_types=[
            pltpu.VMEM((2, 5, 80), jnp.int32),
            pltpu.VMEM((2, CH, INT), jnp.bfloat16),
            pltpu.VMEM((2, CH, INT), jnp.bfloat16),
            pltpu.SemaphoreType.DMA, pltpu.SemaphoreType.DMA,
            pltpu.SemaphoreType.DMA, pltpu.SemaphoreType.DMA,
            pltpu.SemaphoreType.DMA, pltpu.SemaphoreType.DMA,
        ],
    )
    def k(xd_hbm, sbf_hbm, idx_hbm, g_hbm, idx_v, rows_v, sbf_v,
          semg0, semg1, sems0, sems1, semw0, semw1):
        semg = (semg0, semg1)
        sems = (sems0, sems1)
        semw = (semw0, semw1)
        wid = lax.axis_index("s") * NC + lax.axis_index("c")
        c0 = wid * CPW

        def load(ci, b):
            pltpu.sync_copy(idx_hbm.at[pl.ds(ci * 5, 5)], idx_v.at[b])
            for j in range(5):
                pltpu.async_copy(xd_hbm.at[idx_v.at[b, j]],
                                 rows_v.at[b, pl.ds(j * 80, 80)], semg[b])
            pltpu.async_copy(sbf_hbm.at[pl.ds(ci * CH, CH)], sbf_v.at[b], sems[b])

        def drain_w(b):
            for q in range(QN):
                pltpu.make_async_copy(
                    rows_v.at[b, :, pl.ds(q * QW, QW)],
                    g_hbm.at[q, pl.ds(0, CH)], semw[b]).wait()

        def process(ci, b):
            for j in range(5):
                pltpu.make_async_copy(
                    xd_hbm.at[pl.ds(0, 80)],
                    rows_v.at[b, pl.ds(j * 80, 80)], semg[b]).wait()
            pltpu.make_async_copy(
                sbf_hbm.at[pl.ds(0, CH)], sbf_v.at[b], sems[b]).wait()

            def mulrow(r, c2):
                for j in range(INT // 32):
                    sl = pl.ds(j * 32, 32)
                    rows_v[b, r, sl] = rows_v[b, r, sl] * sbf_v[b, r, sl]
                return c2

            lax.fori_loop(0, CH, mulrow, 0)
            off = ci * CH
            for q in range(QN):
                pltpu.async_copy(rows_v.at[b, :, pl.ds(q * QW, QW)],
                                 g_hbm.at[q, pl.ds(off, CH)], semw[b])

        load(c0, 0)
        npairs = CPW // 2

        def pair(i, carry):
            @pl.when(i > 0)
            def _():
                drain_w(1)
            load(c0 + 2 * i + 1, 1)
            process(c0 + 2 * i, 0)

            @pl.when(i + 1 < npairs)
            def _():
                drain_w(0)
                load(c0 + 2 * i + 2, 0)
            process(c0 + 2 * i + 1, 1)
            return carry

        lax.fori_loop(0, npairs, pair, 0)
        drain_w(0)
        drain_w(1)

    return k(xd, sbf_t, idx2d)


def _sc_scatter(g, idx2d):
    mesh = plsc.VectorSubcoreMesh(core_axis_name="c", subcore_axis_name="s")

    @functools.partial(
        pl.kernel,
        out_type=jax.ShapeDtypeStruct((E, INT), jnp.bfloat16),
        mesh=mesh,
        compiler_params=pltpu.CompilerParams(use_tc_tiling_on_sc=False),
        scratch_types=[
            pltpu.VMEM((ZRQ, QW), jnp.bfloat16),
            pltpu.VMEM((ZRQ, QW), jnp.bfloat16),
            pltpu.VMEM((2, NJ, 80), jnp.int32),
            pltpu.VMEM((2, NJ, 80), jnp.int32),
            pltpu.VMEM((2, CHS, QW), jnp.bfloat16),
            pltpu.VMEM_SHARED((RQ + 8, QW), jnp.bfloat16),
            pltpu.SemaphoreType.DMA, pltpu.SemaphoreType.DMA,
            pltpu.SemaphoreType.DMA, pltpu.SemaphoreType.DMA,
        ],
    )
    def k(g_hbm, j_hbm, out_hbm, zbuf, stage, jbuf, libuf, vbuf, acc,
          semv0, semv1, sema0, sema1):
        semv = (semv0, semv1)
        sema = (sema0, sema1)
        c = lax.axis_index("c")
        s = lax.axis_index("s")
        crow = c * RQ
        row0 = s * TROWSQ
        zv = jnp.zeros((QW,), jnp.bfloat16)

        def zrow(r, carry):
            zbuf[r, pl.ds(0, QW)] = zv
            return carry

        lax.fori_loop(0, ZRQ, zrow, 0)

        for q in range(QN):
            for i in range(TROWSQ // ZRQ):
                pltpu.sync_copy(zbuf, acc.at[pl.ds(row0 + i * ZRQ, ZRQ)])
            plsc.subcore_barrier()

            def load(ci, b):
                off = ci * CHS
                pltpu.sync_copy(j_hbm.at[pl.ds(ci * NJ, NJ)], jbuf.at[b])
                pltpu.async_copy(g_hbm.at[q, pl.ds(off, CHS)], vbuf.at[b], semv[b])

            def drain_a(b):
                for a in range(NJ):
                    pltpu.make_async_copy(
                        vbuf.at[b, pl.ds(a * 80, 80)],
                        acc.at[pl.ds(0, 80)], sema[b]).wait()

            def process(ci, b):
                for a in range(NJ):
                    for bb in range(5):
                        sl = pl.ds(bb * 16, 16)
                        li = jbuf[b, a, sl] - crow
                        ok = (li >= 0) & (li < RQ)
                        libuf[b, a, sl] = jnp.where(ok, li, RQ)
                pltpu.make_async_copy(
                    g_hbm.at[q, pl.ds(0, CHS)], vbuf.at[b], semv[b]).wait()
                for a in range(NJ):
                    pltpu.async_copy(vbuf.at[b, pl.ds(a * 80, 80)],
                                     acc.at[libuf.at[b, a]], sema[b], add=True)

            ch0 = s * CPT
            load(ch0, 0)
            npairs = CPT // 2

            def pair(i, carry):
                @pl.when(i > 0)
                def _():
                    drain_a(1)
                load(ch0 + 2 * i + 1, 1)
                process(ch0 + 2 * i, 0)

                @pl.when(i + 1 < npairs)
                def _():
                    drain_a(0)
                    load(ch0 + 2 * i + 2, 0)
                process(ch0 + 2 * i + 1, 1)
                return carry

            lax.fori_loop(0, npairs, pair, 0)
            drain_a(0)
            drain_a(1)
            plsc.subcore_barrier()
            for i in range(TROWSQ // ZRQ):
                r0 = row0 + i * ZRQ
                pltpu.sync_copy(acc.at[pl.ds(r0, ZRQ)], stage)
                pltpu.sync_copy(
                    stage,
                    out_hbm.at[pl.ds(crow + r0, ZRQ), pl.ds(q * QW, QW)])

    return k(g, idx2d)


def _back_body(seg_ref, xji_ref, x_ref, wup_ref,
               b0w1_ref, b0b1_ref, b0w2_ref, b0b2_ref,
               linw_ref, linb_ref,
               a0w1_ref, a0b1_ref, a0w2_ref, a0b2_ref,
               a1w1_ref, a1b1_ref, a1w2_ref, a1b2_ref,
               out_ref):
    u = _silu(jnp.dot(seg_ref[...].astype(jnp.float32), wup_ref[...]))
    h = xji_ref[...] + u
    h = h + _silu(jnp.dot(_silu(jnp.dot(h, b0w1_ref[...]) + b0b1_ref[...]),
                          b0w2_ref[...]) + b0b2_ref[...])
    h = _silu(jnp.dot(h, linw_ref[...]) + linb_ref[...]) + x_ref[...]
    h = h + _silu(jnp.dot(_silu(jnp.dot(h, a0w1_ref[...]) + a0b1_ref[...]),
                          a0w2_ref[...]) + a0b2_ref[...])
    h = h + _silu(jnp.dot(_silu(jnp.dot(h, a1w1_ref[...]) + a1b1_ref[...]),
                          a1w2_ref[...]) + a1b2_ref[...])
    out_ref[...] = h


def _run_back(seg, xji, x, p):
    grid = E // BE
    full = lambda s: pl.BlockSpec(s, lambda i: (0, 0))
    w = lambda: full((H, H))
    b = lambda: full((1, H))
    return pl.pallas_call(
        _back_body,
        grid=(grid,),
        in_specs=[
            pl.BlockSpec((BE, INT), lambda i: (i, 0)),
            pl.BlockSpec((BE, H), lambda i: (i, 0)),
            pl.BlockSpec((BE, H), lambda i: (i, 0)),
            full((INT, H)),
            w(), b(), w(), b(),
            w(), b(),
            w(), b(), w(), b(),
            w(), b(), w(), b(),
        ],
        out_specs=pl.BlockSpec((BE, H), lambda i: (i, 0)),
        out_shape=jax.ShapeDtypeStruct((E, H), jnp.float32),
    )(seg, xji, x, p['wup'],
      p['res_b0_w1'], p['res_b0_b1'].reshape(1, H), p['res_b0_w2'], p['res_b0_b2'].reshape(1, H),
      p['lin_w'], p['lin_b'].reshape(1, H),
      p['res_a0_w1'], p['res_a0_b1'].reshape(1, H), p['res_a0_w2'], p['res_a0_b2'].reshape(1, H),
      p['res_a1_w1'], p['res_a1_b1'].reshape(1, H), p['res_a1_w2'], p['res_a1_b2'].reshape(1, H))


def kernel(x, rbf, sbf, idx_kj, idx_ji, params):
    p = params
    xji, xd = _run_front(x, rbf, p)
    sbf_t = _run_sbf(sbf, p)
    idxkj2 = idx_kj.astype(jnp.int32).reshape(T // 80, 80)
    idxji2 = idx_ji.astype(jnp.int32).reshape(T // 80, 80)
    g = _sc_gather_mul(xd, sbf_t, idxkj2)
    segp = _sc_scatter(g, idxji2)
    return _run_back(segp, xji, x, p)


# speedup vs baseline: 7.2906x; 1.2390x over previous
import functools

import jax
import jax.numpy as jnp
import numpy as np
from jax import lax
from jax.experimental import pallas as pl
from jax.experimental.pallas import tpu as pltpu
from jax.experimental.pallas import tpu_sc as plsc

E = 160000
T = 640000
H = 256
INT = 64

BE = 2000
BTS = 3200

NC = 2
NS = 16
NW = NC * NS

CH = 400
NCHUNK = T // CH
CPW = NCHUNK // NW

QN = 2
QW = INT // QN
RQ = 80000
TROWSQ = RQ // NS
ZRQ = 250
CHS = 800
NCHUNK_S = T // CHS
CPT = NCHUNK_S // NS
NJ = CHS // 80


def _silu(v):
    return v * jax.nn.sigmoid(v)


def _front_body(x_ref, rbf_ref, wji_ref, bji_ref, wkj_ref, bkj_ref,
                rw1_ref, rw2_ref, wdown_ref, xji_ref, xd_ref):
    x = x_ref[...]
    xji_ref[...] = _silu(jnp.dot(x, wji_ref[...]) + bji_ref[...])
    wr = jnp.dot(rw1_ref[...], rw2_ref[...])
    rbft = jnp.dot(rbf_ref[...], wr)
    t = _silu(jnp.dot(x, wkj_ref[...]) + bkj_ref[...]) * rbft
    xd_ref[...] = _silu(jnp.dot(t, wdown_ref[...])).astype(jnp.bfloat16)


def _run_front(x, rbf, p):
    nr = rbf.shape[1]
    grid = E // BE
    full = lambda s: pl.BlockSpec(s, lambda i: (0, 0))
    return pl.pallas_call(
        _front_body,
        grid=(grid,),
        in_specs=[
            pl.BlockSpec((BE, H), lambda i: (i, 0)),
            pl.BlockSpec((BE, nr), lambda i: (i, 0)),
            full((H, H)), full((1, H)), full((H, H)), full((1, H)),
            full((nr, 8)), full((8, H)), full((H, INT)),
        ],
        out_specs=[
            pl.BlockSpec((BE, H), lambda i: (i, 0)),
            pl.BlockSpec((BE, INT), lambda i: (i, 0)),
        ],
        out_shape=[
            jax.ShapeDtypeStruct((E, H), jnp.float32),
            jax.ShapeDtypeStruct((E, INT), jnp.bfloat16),
        ],
    )(x, rbf, p['wji'], p['bji'].reshape(1, H), p['wkj'], p['bkj'].reshape(1, H),
      p['rbf_w1'], p['rbf_w2'], p['wdown'])


def _sbf_body(s1_ref, s2_ref, w1_ref, w2p_ref, out_ref):
    w = jnp.dot(w1_ref[...], w2p_ref[...])
    dn = (((0,), (0,)), ((), ()))
    out_ref[:, 0:INT] = lax.dot_general(s1_ref[...], w, dn)
    out_ref[:, INT:2 * INT] = lax.dot_general(s2_ref[...], w, dn)


def _run_sbf(sbf_t_in, w1, w2p):
    ns = sbf_t_in.shape[0]
    half_blocks = (T // 2) // BTS
    return pl.pallas_call(
        _sbf_body,
        grid=(half_blocks,),
        in_specs=[
            pl.BlockSpec((ns, BTS), lambda i: (0, i)),
            pl.BlockSpec((ns, BTS), lambda i: (0, i + half_blocks)),
            pl.BlockSpec((ns, 8), lambda i: (0, 0)),
            pl.BlockSpec((8, INT), lambda i: (0, 0)),
        ],
        out_specs=pl.BlockSpec((BTS, 2 * INT), lambda i: (i, 0)),
        out_shape=jax.ShapeDtypeStruct((T // 2, 2 * INT), jnp.float32),
    )(sbf_t_in, sbf_t_in, w1, w2p)


def _sc_gather_mul(xd, sbf_t, idx2d):
    mesh = plsc.VectorSubcoreMesh(core_axis_name="c", subcore_axis_name="s")

    @functools.partial(
        pl.kernel,
        out_type=jax.ShapeDtypeStruct((QN, T, QW), jnp.bfloat16),
        mesh=mesh,
        compiler_params=pltpu.CompilerParams(use_tc_tiling_on_sc=False, needs_layout_passes=False),
        scratch_types=[
            pltpu.VMEM((2, 5, 80), jnp.int32),
            pltpu.VMEM((2, CH, INT), jnp.bfloat16),
            pltpu.VMEM((2, CH, INT), jnp.float32),
            pltpu.SemaphoreType.DMA, pltpu.SemaphoreType.DMA,
            pltpu.SemaphoreType.DMA, pltpu.SemaphoreType.DMA,
            pltpu.SemaphoreType.DMA, pltpu.SemaphoreType.DMA,
        ],
    )
    def k(xd_hbm, sbf_hbm, idx_hbm, g_hbm, idx_v, rows_v, sbf_v,
          semg0, semg1, sems0, sems1, semw0, semw1):
        semg = (semg0, semg1)
        sems = (sems0, sems1)
        semw = (semw0, semw1)
        wid = lax.axis_index("s") * NC + lax.axis_index("c")
        c0 = wid * CPW
        half = wid // (NW // 2)
        colb = half * INT
        rowshift = half * (T // 2)

        def load(ci, b):
            pltpu.sync_copy(idx_hbm.at[pl.ds(ci * 5, 5)], idx_v.at[b])
            for j in range(5):
                pltpu.async_copy(xd_hbm.at[idx_v.at[b, j]],
                                 rows_v.at[b, pl.ds(j * 80, 80)], semg[b])
            pltpu.async_copy(
                sbf_hbm.at[pl.ds(ci * CH - rowshift, CH), pl.ds(colb, INT)],
                sbf_v.at[b], sems[b])

        def drain_w(b):
            for q in range(QN):
                pltpu.make_async_copy(
                    rows_v.at[b, :, pl.ds(q * QW, QW)],
                    g_hbm.at[q, pl.ds(0, CH)], semw[b]).wait()

        def process(ci, b):
            for j in range(5):
                pltpu.make_async_copy(
                    xd_hbm.at[pl.ds(0, 80)],
                    rows_v.at[b, pl.ds(j * 80, 80)], semg[b]).wait()
            pltpu.make_async_copy(
                sbf_hbm.at[pl.ds(0, CH), pl.ds(0, INT)], sbf_v.at[b],
                sems[b]).wait()

            def mulrow(r, c2):
                for j in range(INT // 32):
                    se = sbf_v[b, r, pl.ds(j * 32, 16)]
                    so = sbf_v[b, r, pl.ds(j * 32 + 16, 16)]
                    s32 = plsc.pack(se, so, format=plsc.PackFormat.INTERLEAVED)
                    sl = pl.ds(j * 32, 32)
                    rows_v[b, r, sl] = rows_v[b, r, sl] * s32
                return c2

            lax.fori_loop(0, CH, mulrow, 0)
            off = ci * CH
            for q in range(QN):
                pltpu.async_copy(rows_v.at[b, :, pl.ds(q * QW, QW)],
                                 g_hbm.at[q, pl.ds(off, CH)], semw[b])

        load(c0, 0)
        npairs = CPW // 2

        def pair(i, carry):
            @pl.when(i > 0)
            def _():
                drain_w(1)
            load(c0 + 2 * i + 1, 1)
            process(c0 + 2 * i, 0)

            @pl.when(i + 1 < npairs)
            def _():
                drain_w(0)
                load(c0 + 2 * i + 2, 0)
            process(c0 + 2 * i + 1, 1)
            return carry

        lax.fori_loop(0, npairs, pair, 0)
        drain_w(0)
        drain_w(1)

    return k(xd, sbf_t, idx2d)


def _sc_scatter(g, idx2d):
    mesh = plsc.VectorSubcoreMesh(core_axis_name="c", subcore_axis_name="s")

    @functools.partial(
        pl.kernel,
        out_type=jax.ShapeDtypeStruct((E, INT), jnp.float32),
        mesh=mesh,
        compiler_params=pltpu.CompilerParams(use_tc_tiling_on_sc=False, needs_layout_passes=False),
        scratch_types=[
            pltpu.VMEM((ZRQ, QW), jnp.bfloat16),
            pltpu.VMEM((ZRQ, QW), jnp.bfloat16),
            pltpu.VMEM((ZRQ, QW), jnp.float32),
            pltpu.VMEM((2, NJ, 80), jnp.int32),
            pltpu.VMEM((2, NJ, 80), jnp.int32),
            pltpu.VMEM((2, CHS, QW), jnp.bfloat16),
            pltpu.VMEM_SHARED((RQ + 8, QW), jnp.bfloat16),
            pltpu.SemaphoreType.DMA, pltpu.SemaphoreType.DMA,
            pltpu.SemaphoreType.DMA, pltpu.SemaphoreType.DMA,
        ],
    )
    def k(g_hbm, j_hbm, out_hbm, zbuf, bstage, stage, jbuf, libuf, vbuf, acc,
          semv0, semv1, sema0, sema1):
        semv = (semv0, semv1)
        sema = (sema0, sema1)
        c = lax.axis_index("c")
        s = lax.axis_index("s")
        crow = c * RQ
        row0 = s * TROWSQ
        zv = jnp.zeros((QW,), jnp.bfloat16)

        def zrow(r, carry):
            zbuf[r, pl.ds(0, QW)] = zv
            return carry

        lax.fori_loop(0, ZRQ, zrow, 0)

        for q in range(QN):
            for i in range(TROWSQ // ZRQ):
                pltpu.sync_copy(zbuf, acc.at[pl.ds(row0 + i * ZRQ, ZRQ)])
            plsc.subcore_barrier()

            def load(ci, b):
                off = ci * CHS
                pltpu.sync_copy(j_hbm.at[pl.ds(ci * NJ, NJ)], jbuf.at[b])
                pltpu.async_copy(g_hbm.at[q, pl.ds(off, CHS)], vbuf.at[b], semv[b])

            def drain_a(b):
                for a in range(NJ):
                    pltpu.make_async_copy(
                        vbuf.at[b, pl.ds(a * 80, 80)],
                        acc.at[pl.ds(0, 80)], sema[b]).wait()

            def process(ci, b):
                for a in range(NJ):
                    for bb in range(5):
                        sl = pl.ds(bb * 16, 16)
                        li = jbuf[b, a, sl] - crow
                        ok = (li >= 0) & (li < RQ)
                        libuf[b, a, sl] = jnp.where(ok, li, RQ)
                pltpu.make_async_copy(
                    g_hbm.at[q, pl.ds(0, CHS)], vbuf.at[b], semv[b]).wait()
                for a in range(NJ):
                    pltpu.async_copy(vbuf.at[b, pl.ds(a * 80, 80)],
                                     acc.at[libuf.at[b, a]], sema[b], add=True)

            ch0 = s * CPT
            load(ch0, 0)
            npairs = CPT // 2

            def pair(i, carry):
                @pl.when(i > 0)
                def _():
                    drain_a(1)
                load(ch0 + 2 * i + 1, 1)
                process(ch0 + 2 * i, 0)

                @pl.when(i + 1 < npairs)
                def _():
                    drain_a(0)
                    load(ch0 + 2 * i + 2, 0)
                process(ch0 + 2 * i + 1, 1)
                return carry

            lax.fori_loop(0, npairs, pair, 0)
            drain_a(0)
            drain_a(1)
            plsc.subcore_barrier()
            for i in range(TROWSQ // ZRQ):
                r0 = row0 + i * ZRQ
                pltpu.sync_copy(acc.at[pl.ds(r0, ZRQ)], bstage)

                def upc(r, carry):
                    v32 = bstage[r, pl.ds(0, QW)]
                    va, vb = plsc.unpack(v32, format=plsc.PackFormat.INTERLEAVED)
                    stage[r, pl.ds(0, 16)] = va
                    stage[r, pl.ds(16, 16)] = vb
                    return carry

                lax.fori_loop(0, ZRQ, upc, 0)
                pltpu.sync_copy(
                    stage,
                    out_hbm.at[pl.ds(crow + r0, ZRQ), pl.ds(q * QW, QW)])

    return k(g, idx2d)


def _back_body(seg_ref, xji_ref, x_ref, wup_ref,
               b0w1_ref, b0b1_ref, b0w2_ref, b0b2_ref,
               linw_ref, linb_ref,
               a0w1_ref, a0b1_ref, a0w2_ref, a0b2_ref,
               a1w1_ref, a1b1_ref, a1w2_ref, a1b2_ref,
               out_ref):
    u = _silu(jnp.dot(seg_ref[...], wup_ref[...]))
    h = xji_ref[...] + u
    h = h + _silu(jnp.dot(_silu(jnp.dot(h, b0w1_ref[...]) + b0b1_ref[...]),
                          b0w2_ref[...]) + b0b2_ref[...])
    h = _silu(jnp.dot(h, linw_ref[...]) + linb_ref[...]) + x_ref[...]
    h = h + _silu(jnp.dot(_silu(jnp.dot(h, a0w1_ref[...]) + a0b1_ref[...]),
                          a0w2_ref[...]) + a0b2_ref[...])
    h = h + _silu(jnp.dot(_silu(jnp.dot(h, a1w1_ref[...]) + a1b1_ref[...]),
                          a1w2_ref[...]) + a1b2_ref[...])
    out_ref[...] = h


def _run_back(seg, xji, x, p, wup_p):
    grid = E // BE
    full = lambda s: pl.BlockSpec(s, lambda i: (0, 0))
    w = lambda: full((H, H))
    b = lambda: full((1, H))
    return pl.pallas_call(
        _back_body,
        grid=(grid,),
        in_specs=[
            pl.BlockSpec((BE, INT), lambda i: (i, 0)),
            pl.BlockSpec((BE, H), lambda i: (i, 0)),
            pl.BlockSpec((BE, H), lambda i: (i, 0)),
            full((INT, H)),
            w(), b(), w(), b(),
            w(), b(),
            w(), b(), w(), b(),
            w(), b(), w(), b(),
        ],
        out_specs=pl.BlockSpec((BE, H), lambda i: (i, 0)),
        out_shape=jax.ShapeDtypeStruct((E, H), jnp.float32),
    )(seg, xji, x, wup_p,
      p['res_b0_w1'], p['res_b0_b1'].reshape(1, H), p['res_b0_w2'], p['res_b0_b2'].reshape(1, H),
      p['lin_w'], p['lin_b'].reshape(1, H),
      p['res_a0_w1'], p['res_a0_b1'].reshape(1, H), p['res_a0_w2'], p['res_a0_b2'].reshape(1, H),
      p['res_a1_w1'], p['res_a1_b1'].reshape(1, H), p['res_a1_w2'], p['res_a1_b2'].reshape(1, H))


_PERM = np.concatenate([np.arange(0, 32, 2), np.arange(1, 32, 2),
                        np.arange(32, 64, 2), np.arange(33, 64, 2)])


def kernel(x, rbf, sbf, idx_kj, idx_ji, params):
    p = params
    xji, xd = _run_front(x, rbf, p)
    sbf2 = _run_sbf(jnp.transpose(sbf), p['sbf_w1'], p['sbf_w2'][:, _PERM])
    idxkj2 = idx_kj.astype(jnp.int32).reshape(T // 80, 80)
    idxji2 = idx_ji.astype(jnp.int32).reshape(T // 80, 80)
    g = _sc_gather_mul(xd, sbf2, idxkj2)
    seg = _sc_scatter(g, idxji2)
    return _run_back(seg, xji, x, p, p['wup'][_PERM, :])
